# baseline (speedup 1.0000x reference)
import jax
import jax.numpy as jnp
import numpy as np
from jax import lax
from jax.experimental import pallas as pl

N_BINS = 65
N_Q = 66
N_R = 2 * N_Q
APAD = 128
NBKT = 256
CPB = 8
TOP = np.int32(-2147483648)


def _sortable_key(xbits):
    return jnp.where(xbits >= 0, xbits ^ TOP, ~xbits)


def _byte(u, p):
    return lax.shift_right_logical(u, 24 - 8 * p) & 255


def _one_channel(xr, xc, qv, L):
    f32 = jnp.float32
    bf16 = jnp.bfloat16

    iota_r = lax.broadcasted_iota(jnp.int32, (1, L), 1)

    nzr = xr != 0.0
    start = jnp.min(jnp.where(nzr, iota_r, L))
    end = jnp.max(jnp.where(nzr, iota_r, -1))
    cnt = jnp.maximum(end - start + 1, 0)
    maxcnt = jnp.maximum(cnt - 1, 0).astype(f32)

    validr = (iota_r >= start) & (iota_r <= end)
    validr_bf = jnp.where(validr, 1.0, 0.0).astype(bf16)

    posq = qv * maxcnt
    lo = jnp.floor(posq)
    frac = posq - lo
    kcol = lo

    u_r = _sortable_key(lax.bitcast_convert_type(xr, jnp.int32))
    u_c = _sortable_key(lax.bitcast_convert_type(xc, jnp.int32))

    def thermo_col(sh, nb):
        dig = lax.shift_right_logical(u_c, sh) & (nb - 1)
        return (dig <= lax.broadcasted_iota(jnp.int32, (L, nb), 1)).astype(bf16)

    def counts(mask_bf, sh, nb):
        return lax.dot_general(mask_bf, thermo_col(sh, nb),
                               (((1,), (0,)), ((), ())),
                               preferred_element_type=f32)

    um = jnp.where(validr, u_r, np.int32(-1))

    digits = [(24, 256), (16, 256), (8, 256), (0, 256)]
    mbf = validr_bf
    r = kcol
    usel = jnp.zeros((N_Q, 1), jnp.int32)
    inc = None
    for i, (sh, nb) in enumerate(digits):
        sp = counts(mbf, sh, nb)
        bm = sp <= r
        dsel = jnp.sum(bm.astype(f32), axis=1, keepdims=True)
        usel = usel | lax.shift_left(dsel.astype(jnp.int32), sh)
        if i < len(digits) - 1:
            r = r - jnp.max(jnp.where(bm, sp, 0.0), axis=1, keepdims=True)
            mbf = (lax.shift_right_logical(um, sh)
                   == lax.shift_right_logical(usel, sh)).astype(bf16)
        else:
            inc = jnp.min(jnp.where(bm, jnp.float32(4097.0), sp),
                          axis=1, keepdims=True)

    sbits = jnp.where(usel < 0, usel ^ TOP, ~usel)
    vlo = lax.bitcast_convert_type(sbits, jnp.float32)

    keys_r = u_r ^ TOP
    keysel = usel ^ TOP
    big = jnp.float32(jnp.finfo(jnp.float32).max)
    nxt_cand = jnp.where((keys_r > keysel) & validr, xr, big)
    minnext = jnp.min(nxt_cand, axis=1, keepdims=True)
    vhi = jnp.where((inc >= r + 2.0) | (minnext >= big), vlo, minnext)
    edges = vlo + (vhi - vlo) * frac
    edges = jnp.where(cnt > 0, edges, 0.0)

    e65 = edges[0:N_BINS, :]
    cb = jnp.where(xr >= e65, 1.0, 0.0).astype(bf16)
    ones_row = jnp.full((1, L), 1.0, bf16)
    hi_rows = jnp.concatenate([ones_row, cb[0:N_BINS - 1, :]], axis=0)
    lo_rows = jnp.concatenate([cb[0:N_BINS - 1, :],
                               jnp.zeros((1, L), bf16)], axis=0)
    onehot = hi_rows - lo_rows
    tmask_bf = jnp.where(iota_r < (L - 1), 1.0, 0.0).astype(bf16)
    oc = onehot * tmask_bf
    on = jnp.concatenate([onehot[:, 1:L], onehot[:, 0:1]], axis=1)
    m = lax.dot_general(oc, on, (((1,), (1,)), ((), ())),
                        preferred_element_type=f32)
    return m * (1.0 / (L - 1))


def _mtf_body(x_ref, xt_ref, q_ref, o_ref):
    L = x_ref.shape[2]
    qv = q_ref[...]
    for c in range(CPB):
        o_ref[c, :, :] = _one_channel(x_ref[c], xt_ref[c], qv, L)


def kernel(x):
    N, C, L = x.shape
    nc = N * C
    xf = x.reshape(nc, 1, L)
    xt = x.reshape(nc, L, 1)
    q = jnp.linspace(0.0, 1.0, N_Q, dtype=jnp.float32).reshape(N_Q, 1)

    out = pl.pallas_call(
        _mtf_body,
        grid=(nc // CPB,),
        in_specs=[
            pl.BlockSpec((CPB, 1, L), lambda i: (i, 0, 0)),
            pl.BlockSpec((CPB, L, 1), lambda i: (i, 0, 0)),
            pl.BlockSpec((N_Q, 1), lambda i: (0, 0)),
        ],
        out_specs=pl.BlockSpec((CPB, N_BINS, N_BINS), lambda i: (i, 0, 0)),
        out_shape=jax.ShapeDtypeStruct((nc, N_BINS, N_BINS), jnp.float32),
    )(xf, xt, q)
    return out.reshape(N, C, N_BINS, N_BINS)


# speedup vs baseline: 1.0196x; 1.0196x over previous
import jax
import jax.numpy as jnp
import numpy as np
from jax import lax
from jax.experimental import pallas as pl

N_BINS = 65
N_Q = 66
CPB = 4
TOP = np.int32(-2147483648)


def _sortable_key(xbits):
    return jnp.where(xbits >= 0, xbits ^ TOP, ~xbits)


def _one_channel(xr, xc, qv, L):
    f32 = jnp.float32
    bf16 = jnp.bfloat16

    iota_r = lax.broadcasted_iota(jnp.int32, (1, L), 1)

    nzr = xr != 0.0
    start = jnp.min(jnp.where(nzr, iota_r, L))
    end = jnp.max(jnp.where(nzr, iota_r, -1))
    cnt = jnp.maximum(end - start + 1, 0)
    maxcnt = jnp.maximum(cnt - 1, 0).astype(f32)

    validr = (iota_r >= start) & (iota_r <= end)
    validr_bf = jnp.where(validr, 1.0, 0.0).astype(bf16)

    posq = qv * maxcnt
    lo = jnp.floor(posq)
    frac = posq - lo
    kcol = lo

    u_r = _sortable_key(lax.bitcast_convert_type(xr, jnp.int32))
    u_c = _sortable_key(lax.bitcast_convert_type(xc, jnp.int32))

    def thermo_col(sh, nb):
        dig = lax.shift_right_logical(u_c, sh) & (nb - 1)
        return (dig <= lax.broadcasted_iota(jnp.int32, (L, nb), 1)).astype(bf16)

    def counts(mask_bf, sh, nb):
        return lax.dot_general(mask_bf, thermo_col(sh, nb),
                               (((1,), (0,)), ((), ())),
                               preferred_element_type=f32)

    um = jnp.where(validr, u_r, np.int32(-1))

    digits = [(24, 256), (16, 256), (8, 256), (0, 256)]
    mbf = validr_bf
    r = kcol
    usel = jnp.zeros((N_Q, 1), jnp.int32)
    inc = None
    for i, (sh, nb) in enumerate(digits):
        sp = counts(mbf, sh, nb)
        bm = sp <= r
        dsel = jnp.sum(bm.astype(f32), axis=1, keepdims=True)
        usel = usel | lax.shift_left(dsel.astype(jnp.int32), sh)
        if i < len(digits) - 1:
            r = r - jnp.max(jnp.where(bm, sp, 0.0), axis=1, keepdims=True)
            mbf = (lax.shift_right_logical(um, sh)
                   == lax.shift_right_logical(usel, sh)).astype(bf16)
        else:
            inc = jnp.min(jnp.where(bm, jnp.float32(4097.0), sp),
                          axis=1, keepdims=True)

    sbits = jnp.where(usel < 0, usel ^ TOP, ~usel)
    vlo = lax.bitcast_convert_type(sbits, jnp.float32)

    keys_r = u_r ^ TOP
    keysel = usel ^ TOP
    big = jnp.float32(jnp.finfo(jnp.float32).max)
    nxt_cand = jnp.where((keys_r > keysel) & validr, xr, big)
    minnext = jnp.min(nxt_cand, axis=1, keepdims=True)
    vhi = jnp.where((inc >= r + 2.0) | (minnext >= big), vlo, minnext)
    edges = vlo + (vhi - vlo) * frac
    edges = jnp.where(cnt > 0, edges, 0.0)

    e65 = edges[0:N_BINS, :]
    cb = jnp.where(xr >= e65, 1.0, 0.0).astype(bf16)
    ones_row = jnp.full((1, L), 1.0, bf16)
    hi_rows = jnp.concatenate([ones_row, cb[0:N_BINS - 1, :]], axis=0)
    lo_rows = jnp.concatenate([cb[0:N_BINS - 1, :],
                               jnp.zeros((1, L), bf16)], axis=0)
    onehot = hi_rows - lo_rows
    tmask_bf = jnp.where(iota_r < (L - 1), 1.0, 0.0).astype(bf16)
    oc = onehot * tmask_bf
    on = jnp.concatenate([onehot[:, 1:L], onehot[:, 0:1]], axis=1)
    m = lax.dot_general(oc, on, (((1,), (1,)), ((), ())),
                        preferred_element_type=f32)
    return m * (1.0 / (L - 1))


def _mtf_body(x_ref, xt_ref, q_ref, o_ref):
    L = x_ref.shape[2]
    qv = q_ref[...]
    for c in range(CPB):
        o_ref[c, :, :] = _one_channel(x_ref[c], xt_ref[c], qv, L)


def kernel(x):
    N, C, L = x.shape
    nc = N * C
    xf = x.reshape(nc, 1, L)
    xt = x.reshape(nc, L, 1)
    q = jnp.linspace(0.0, 1.0, N_Q, dtype=jnp.float32).reshape(N_Q, 1)

    out = pl.pallas_call(
        _mtf_body,
        grid=(nc // CPB,),
        in_specs=[
            pl.BlockSpec((CPB, 1, L), lambda i: (i, 0, 0)),
            pl.BlockSpec((CPB, L, 1), lambda i: (i, 0, 0)),
            pl.BlockSpec((N_Q, 1), lambda i: (0, 0)),
        ],
        out_specs=pl.BlockSpec((CPB, N_BINS, N_BINS), lambda i: (i, 0, 0)),
        out_shape=jax.ShapeDtypeStruct((nc, N_BINS, N_BINS), jnp.float32),
    )(xf, xt, q)
    return out.reshape(N, C, N_BINS, N_BINS)
